# Initial kernel scaffold; baseline (speedup 1.0000x reference)
import functools

import jax
import jax.numpy as jnp
from jax import lax
from jax.experimental import pallas as pl
from jax.experimental.pallas import tpu as pltpu
from jax.experimental.pallas import tpu_sc as plsc

N = 10000
H = 160
R = 50
NB = 5
BD = 32
E = 320000

NC = 2
NS = 16
NW = NC * NS

EPW = E // NW
CH = 80
NCH = EPW // CH

CNCH = E // (NS * CH)

NRP = 512000
CSPAN = NRP // NS
CZCH = 4000

NPS = N // NS
ZRCH = 25

NT = 10
TN = N // NT


def _cnt_body(cidx_hbm, invcnt_hbm, cidx_v, buf_v, ones_v, cnt_sh):
  c = lax.axis_index("c")
  s = lax.axis_index("s")

  @pl.when(c == 0)
  def _zero():
    def zb(i, _):
      buf_v[pl.ds(i * 16, 16)] = jnp.zeros((16,), jnp.float32)
      return 0
    lax.fori_loop(0, CZCH // 16, zb, 0)
    base = s * CSPAN
    for k in range(CSPAN // CZCH):
      pltpu.sync_copy(buf_v, cnt_sh.at[pl.ds(base + k * CZCH, CZCH)])

  plsc.subcore_barrier()

  @pl.when(c == 0)
  def _count():
    for q in range(CH // 16):
      ones_v[pl.ds(q * 16, 16)] = jnp.ones((16,), jnp.float32)
    pltpu.sync_copy(cidx_hbm.at[s], cidx_v)

    def body(j, _):
      pltpu.sync_copy(ones_v, cnt_sh.at[cidx_v.at[j]], add=True)
      return 0
    lax.fori_loop(0, CNCH, body, 0)

  plsc.subcore_barrier()

  @pl.when(c == 0)
  def _inv():
    base = s * CSPAN
    for k in range(CSPAN // CZCH):
      pltpu.sync_copy(cnt_sh.at[pl.ds(base + k * CZCH, CZCH)], buf_v)

      def ib(i, _):
        v = buf_v[pl.ds(i * 16, 16)]
        buf_v[pl.ds(i * 16, 16)] = 1.0 / jnp.maximum(v, 1.0)
        return 0
      lax.fori_loop(0, CZCH // 16, ib, 0)
      pltpu.sync_copy(buf_v, invcnt_hbm.at[pl.ds(base + k * CZCH, CZCH)])


_cnt_kernel = pl.kernel(
    _cnt_body,
    out_type=jax.ShapeDtypeStruct((NRP,), jnp.float32),
    mesh=plsc.VectorSubcoreMesh(
        core_axis_name="c", subcore_axis_name="s", num_cores=NC,
        num_subcores=NS),
    scratch_types=[
        pltpu.VMEM((CNCH, CH), jnp.int32),
        pltpu.VMEM((CZCH,), jnp.float32),
        pltpu.VMEM((CH,), jnp.float32),
        pltpu.VMEM_SHARED((NRP,), jnp.float32),
    ],
)


def _agg_body(z_hbm, gidx_hbm, dst_hbm, cidx_hbm, invcnt_hbm, acc_hbm,
              gidx_v, dst_v, cidx_v, rows_v, s_v, zb_v, acc_sh, sem):
  c = lax.axis_index("c")
  s = lax.axis_index("s")
  wid = c * NS + s

  def zb(i, _):
    zb_v[i // 10, pl.ds((i % 10) * 16, 16)] = jnp.zeros((16,), jnp.float32)
    return 0
  lax.fori_loop(0, ZRCH * (H // 16), zb, 0)
  for k in range(NPS // ZRCH):
    pltpu.sync_copy(zb_v, acc_sh.at[pl.ds(s * NPS + k * ZRCH, ZRCH)])

  pltpu.sync_copy(gidx_hbm.at[wid], gidx_v)
  pltpu.sync_copy(dst_hbm.at[wid], dst_v)
  pltpu.sync_copy(cidx_hbm.at[wid], cidx_v)

  plsc.subcore_barrier()

  def body(j, _):
    pltpu.async_copy(z_hbm.at[gidx_v.at[j]], rows_v, sem).wait()
    pltpu.async_copy(invcnt_hbm.at[cidx_v.at[j]], s_v, sem).wait()

    def sb(i, _):
      sv = plsc.load_gather(s_v, [jnp.full((16,), i, jnp.int32)])
      for q in range(H // 16):
        rows_v[i, pl.ds(q * 16, 16)] = rows_v[i, pl.ds(q * 16, 16)] * sv
      return 0
    lax.fori_loop(0, CH, sb, 0)

    pltpu.sync_copy(rows_v, acc_sh.at[dst_v.at[j]], add=True)
    return 0
  lax.fori_loop(0, NCH, body, 0)

  plsc.subcore_barrier()

  pltpu.sync_copy(acc_sh.at[pl.ds(s * NPS, NPS)],
                  acc_hbm.at[c, pl.ds(s * NPS, NPS)])


_agg_kernel = pl.kernel(
    _agg_body,
    out_type=jax.ShapeDtypeStruct((NC, N, H), jnp.float32),
    mesh=plsc.VectorSubcoreMesh(
        core_axis_name="c", subcore_axis_name="s", num_cores=NC,
        num_subcores=NS),
    scratch_types=[
        pltpu.VMEM((NCH, CH), jnp.int32),
        pltpu.VMEM((NCH, CH), jnp.int32),
        pltpu.VMEM((NCH, CH), jnp.int32),
        pltpu.VMEM((CH, H), jnp.float32),
        pltpu.VMEM((CH,), jnp.float32),
        pltpu.VMEM((ZRCH, H), jnp.float32),
        pltpu.VMEM_SHARED((N, H), jnp.float32),
        pltpu.SemaphoreType.DMA,
    ],
)


def _z_body(x_ref, w_ref, z_ref):
  xb = x_ref[...]
  for b in range(NB):
    z_ref[:, b * BD:(b + 1) * BD] = jnp.dot(
        xb[:, b * BD:(b + 1) * BD], w_ref[0, b],
        preferred_element_type=jnp.float32)


def _z_call(x, w):
  return pl.pallas_call(
      _z_body,
      grid=(NT, R),
      in_specs=[
          pl.BlockSpec((TN, H), lambda nt, r: (nt, 0)),
          pl.BlockSpec((1, NB, BD, BD), lambda nt, r: (r, 0, 0, 0)),
      ],
      out_specs=pl.BlockSpec((TN, H), lambda nt, r: (r * NT + nt, 0)),
      out_shape=jax.ShapeDtypeStruct((R * N, H), jnp.float32),
  )(x, w)


def _fin_body(a_ref, x_ref, root_ref, bias_ref, o_ref, *, relu):
  t = (a_ref[0] + a_ref[1]
       + jnp.dot(x_ref[...], root_ref[...],
                 preferred_element_type=jnp.float32)
       + bias_ref[...])
  o_ref[...] = jnp.maximum(t, 0.0) if relu else t


def _fin_call(acc, x, root, bias, relu):
  return pl.pallas_call(
      functools.partial(_fin_body, relu=relu),
      grid=(NT,),
      in_specs=[
          pl.BlockSpec((NC, TN, H), lambda nt: (0, nt, 0)),
          pl.BlockSpec((TN, H), lambda nt: (nt, 0)),
          pl.BlockSpec((H, H), lambda nt: (0, 0)),
          pl.BlockSpec((1, H), lambda nt: (0, 0)),
      ],
      out_specs=pl.BlockSpec((TN, H), lambda nt: (nt, 0)),
      out_shape=jax.ShapeDtypeStruct((N, H), jnp.float32),
  )(acc, x, root, bias)


def kernel(node_emb, edge_index, edge_type, W1, root1, bias1, W2, root2,
           bias2):
  src = edge_index[0].astype(jnp.int32)
  dst = edge_index[1].astype(jnp.int32)
  et = edge_type.astype(jnp.int32)

  gidx = (et * N + src).reshape(NW, NCH, CH)
  dstr = dst.reshape(NW, NCH, CH)
  cidx_flat = dst * R + et
  cidx32 = cidx_flat.reshape(NW, NCH, CH)
  cidx16 = cidx_flat.reshape(NS, CNCH, CH)
  bias1_2d = bias1.reshape(1, H)
  bias2_2d = bias2.reshape(1, H)

  invcnt = _cnt_kernel(cidx16)

  z1 = _z_call(node_emb, W1)
  acc1 = _agg_kernel(z1, gidx, dstr, cidx32, invcnt)
  h1 = _fin_call(acc1, node_emb, root1, bias1_2d, relu=True)

  z2 = _z_call(h1, W2)
  acc2 = _agg_kernel(z2, gidx, dstr, cidx32, invcnt)
  out = _fin_call(acc2, h1, root2, bias2_2d, relu=False)
  return out


# speedup vs baseline: 41.0930x; 41.0930x over previous
import functools

import jax
import jax.numpy as jnp
from jax import lax
from jax.experimental import pallas as pl
from jax.experimental.pallas import tpu as pltpu
from jax.experimental.pallas import tpu_sc as plsc

N = 10000
H = 160
R = 50
NB = 5
BD = 32
E = 320000

NC = 2
NS = 16
NW = NC * NS

EPW = E // NW
CH = 80
NCH = EPW // CH
SCI = 5

CNCH = E // (NS * CH)

NRP = 512000
CSPAN = NRP // NS
CZCH = 4000

NAP = 10240
NPS = NAP // NS

NT = 10
TN = N // NT


def _cnt_body(cidx_hbm, invcnt_hbm, cidx_v, buf_v, ones_v, cnt_sh):
  c = lax.axis_index("c")
  s = lax.axis_index("s")

  @pl.when(c == 0)
  def _zero():
    def zb(i, _):
      buf_v[pl.ds(i * 16, 16)] = jnp.zeros((16,), jnp.float32)
      return 0
    lax.fori_loop(0, CZCH // 16, zb, 0)
    base = s * CSPAN
    for k in range(CSPAN // CZCH):
      pltpu.sync_copy(buf_v, cnt_sh.at[pl.ds(base + k * CZCH, CZCH)])

  plsc.subcore_barrier()

  @pl.when(c == 0)
  def _count():
    for q in range(CH // 16):
      ones_v[pl.ds(q * 16, 16)] = jnp.ones((16,), jnp.float32)
    pltpu.sync_copy(cidx_hbm.at[s], cidx_v)

    def body(j, _):
      pltpu.sync_copy(ones_v, cnt_sh.at[cidx_v.at[j]], add=True)
      return 0
    lax.fori_loop(0, CNCH, body, 0)

  plsc.subcore_barrier()

  @pl.when(c == 0)
  def _inv():
    base = s * CSPAN
    for k in range(CSPAN // CZCH):
      pltpu.sync_copy(cnt_sh.at[pl.ds(base + k * CZCH, CZCH)], buf_v)

      def ib(i, _):
        v = buf_v[pl.ds(i * 16, 16)]
        buf_v[pl.ds(i * 16, 16)] = 1.0 / jnp.maximum(v, 1.0)
        return 0
      lax.fori_loop(0, CZCH // 16, ib, 0)
      pltpu.sync_copy(buf_v, invcnt_hbm.at[pl.ds(base + k * CZCH, CZCH)])


_cnt_kernel = pl.kernel(
    _cnt_body,
    out_type=jax.ShapeDtypeStruct((NRP,), jnp.float32),
    mesh=plsc.VectorSubcoreMesh(
        core_axis_name="c", subcore_axis_name="s", num_cores=NC,
        num_subcores=NS),
    scratch_types=[
        pltpu.VMEM((CNCH, CH), jnp.int32),
        pltpu.VMEM((CZCH,), jnp.float32),
        pltpu.VMEM((CH,), jnp.float32),
        pltpu.VMEM_SHARED((NRP,), jnp.float32),
    ],
    compiler_params=pltpu.CompilerParams(use_tc_tiling_on_sc=False, needs_layout_passes=False),
)


def _agg_body(z_hbm, gidx_hbm, dst_hbm, cidx_hbm, invcnt_hbm, acc_hbm,
              gidx_v, dst_v, cidx_v, rows_v, s_v, acc_sh, sem):
  c = lax.axis_index("c")
  s = lax.axis_index("s")
  wid = c * NS + s

  def zb(i, _):
    rows_v[i // (H // 16), pl.ds((i % (H // 16)) * 16, 16)] = jnp.zeros(
        (16,), jnp.float32)
    return 0
  lax.fori_loop(0, CH * (H // 16), zb, 0)
  for k in range(NPS // CH):
    pltpu.sync_copy(rows_v, acc_sh.at[pl.ds(s * NPS + k * CH, CH)])

  plsc.subcore_barrier()

  def body(sc, _):
    pltpu.sync_copy(gidx_hbm.at[wid, pl.ds(sc * SCI, SCI)], gidx_v)
    pltpu.sync_copy(dst_hbm.at[wid, pl.ds(sc * SCI, SCI)], dst_v)
    pltpu.sync_copy(cidx_hbm.at[wid, pl.ds(sc * SCI, SCI)], cidx_v)
    for k in range(SCI):
      pltpu.async_copy(z_hbm.at[gidx_v.at[k]], rows_v, sem).wait()
      pltpu.async_copy(invcnt_hbm.at[cidx_v.at[k]], s_v, sem).wait()

      def sb(i, _):
        sv = plsc.load_gather(s_v, [jnp.full((16,), i, jnp.int32)])
        for q in range(H // 16):
          rows_v[i, pl.ds(q * 16, 16)] = rows_v[i, pl.ds(q * 16, 16)] * sv
        return 0
      lax.fori_loop(0, CH, sb, 0)

      pltpu.sync_copy(rows_v, acc_sh.at[dst_v.at[k]], add=True)
    return 0
  lax.fori_loop(0, NCH // SCI, body, 0)

  plsc.subcore_barrier()

  pltpu.sync_copy(acc_sh.at[pl.ds(s * NPS, NPS)],
                  acc_hbm.at[c, pl.ds(s * NPS, NPS)])


_agg_kernel = pl.kernel(
    _agg_body,
    out_type=jax.ShapeDtypeStruct((NC, NAP, H), jnp.float32),
    mesh=plsc.VectorSubcoreMesh(
        core_axis_name="c", subcore_axis_name="s", num_cores=NC,
        num_subcores=NS),
    scratch_types=[
        pltpu.VMEM((SCI, CH), jnp.int32),
        pltpu.VMEM((SCI, CH), jnp.int32),
        pltpu.VMEM((SCI, CH), jnp.int32),
        pltpu.VMEM((CH, H), jnp.float32),
        pltpu.VMEM((CH,), jnp.float32),
        pltpu.VMEM_SHARED((NAP, H), jnp.float32),
        pltpu.SemaphoreType.DMA,
    ],
    compiler_params=pltpu.CompilerParams(use_tc_tiling_on_sc=False, needs_layout_passes=False),
)


def _z_body(x_ref, w_ref, z_ref):
  xb = x_ref[...]
  for b in range(NB):
    z_ref[:, b * BD:(b + 1) * BD] = jnp.dot(
        xb[:, b * BD:(b + 1) * BD], w_ref[0, b],
        preferred_element_type=jnp.float32)


def _z_call(x, w):
  return pl.pallas_call(
      _z_body,
      grid=(NT, R),
      in_specs=[
          pl.BlockSpec((TN, H), lambda nt, r: (nt, 0)),
          pl.BlockSpec((1, NB, BD, BD), lambda nt, r: (r, 0, 0, 0)),
      ],
      out_specs=pl.BlockSpec((TN, H), lambda nt, r: (r * NT + nt, 0)),
      out_shape=jax.ShapeDtypeStruct((R * N, H), jnp.float32),
  )(x, w)


def _fin_body(a_ref, x_ref, root_ref, bias_ref, o_ref, *, relu):
  t = (a_ref[0] + a_ref[1]
       + jnp.dot(x_ref[...], root_ref[...],
                 preferred_element_type=jnp.float32)
       + bias_ref[...])
  o_ref[...] = jnp.maximum(t, 0.0) if relu else t


def _fin_call(acc, x, root, bias, relu):
  return pl.pallas_call(
      functools.partial(_fin_body, relu=relu),
      grid=(NT,),
      in_specs=[
          pl.BlockSpec((NC, TN, H), lambda nt: (0, nt, 0)),
          pl.BlockSpec((TN, H), lambda nt: (nt, 0)),
          pl.BlockSpec((H, H), lambda nt: (0, 0)),
          pl.BlockSpec((1, H), lambda nt: (0, 0)),
      ],
      out_specs=pl.BlockSpec((TN, H), lambda nt: (nt, 0)),
      out_shape=jax.ShapeDtypeStruct((N, H), jnp.float32),
  )(acc, x, root, bias)


def kernel(node_emb, edge_index, edge_type, W1, root1, bias1, W2, root2,
           bias2):
  src = edge_index[0].astype(jnp.int32)
  dst = edge_index[1].astype(jnp.int32)
  et = edge_type.astype(jnp.int32)

  gidx = (et * N + src).reshape(NW, NCH, CH)
  dstr = dst.reshape(NW, NCH, CH)
  cidx_flat = dst * R + et
  cidx32 = cidx_flat.reshape(NW, NCH, CH)
  cidx16 = cidx_flat.reshape(NS, CNCH, CH)
  bias1_2d = bias1.reshape(1, H)
  bias2_2d = bias2.reshape(1, H)

  invcnt = _cnt_kernel(cidx16)

  z1 = _z_call(node_emb, W1)
  acc1 = _agg_kernel(z1, gidx, dstr, cidx32, invcnt)
  h1 = _fin_call(acc1, node_emb, root1, bias1_2d, relu=True)

  z2 = _z_call(h1, W2)
  acc2 = _agg_kernel(z2, gidx, dstr, cidx32, invcnt)
  out = _fin_call(acc2, h1, root2, bias2_2d, relu=False)
  return out


# speedup vs baseline: 52.2916x; 1.2725x over previous
import functools

import jax
import jax.numpy as jnp
from jax import lax
from jax.experimental import pallas as pl
from jax.experimental.pallas import tpu as pltpu
from jax.experimental.pallas import tpu_sc as plsc

N = 10000
H = 160
HA = 128
HB = 32
R = 50
NB = 5
BD = 32
E = 320000

NC = 2
NS = 16
NW = NC * NS

EPW = E // NW
CH = 80
NCH = EPW // CH
SCI = 5

CNCH = E // (NS * CH)

NRP = 512000
CSPAN = NRP // NS
CZCH = 4000

NAP = 10240
NPS = NAP // NS

NT = 10
TN = N // NT
ZG = R * N // 8


def _cnt_body(cidx_hbm, invcnt_hbm, cidx_v, buf_v, ones_v, cnt_sh):
  c = lax.axis_index("c")
  s = lax.axis_index("s")

  @pl.when(c == 0)
  def _zero():
    def zb(i, _):
      buf_v[pl.ds(i * 16, 16)] = jnp.zeros((16,), jnp.float32)
      return 0
    lax.fori_loop(0, CZCH // 16, zb, 0)
    base = s * CSPAN
    for k in range(CSPAN // CZCH):
      pltpu.sync_copy(buf_v, cnt_sh.at[pl.ds(base + k * CZCH, CZCH)])

  plsc.subcore_barrier()

  @pl.when(c == 0)
  def _count():
    for q in range(CH // 16):
      ones_v[pl.ds(q * 16, 16)] = jnp.ones((16,), jnp.float32)
    pltpu.sync_copy(cidx_hbm.at[s], cidx_v)

    def body(j, _):
      pltpu.sync_copy(ones_v, cnt_sh.at[cidx_v.at[j]], add=True)
      return 0
    lax.fori_loop(0, CNCH, body, 0)

  plsc.subcore_barrier()

  @pl.when(c == 0)
  def _inv():
    base = s * CSPAN
    for k in range(CSPAN // CZCH):
      pltpu.sync_copy(cnt_sh.at[pl.ds(base + k * CZCH, CZCH)], buf_v)

      def ib(i, _):
        v = buf_v[pl.ds(i * 16, 16)]
        buf_v[pl.ds(i * 16, 16)] = 1.0 / jnp.maximum(v, 1.0)
        return 0
      lax.fori_loop(0, CZCH // 16, ib, 0)
      pltpu.sync_copy(buf_v, invcnt_hbm.at[pl.ds(base + k * CZCH, CZCH)])


_cnt_kernel = pl.kernel(
    _cnt_body,
    out_type=jax.ShapeDtypeStruct((NRP,), jnp.float32),
    mesh=plsc.VectorSubcoreMesh(
        core_axis_name="c", subcore_axis_name="s", num_cores=NC,
        num_subcores=NS),
    scratch_types=[
        pltpu.VMEM((CNCH, CH), jnp.int32),
        pltpu.VMEM((CZCH,), jnp.float32),
        pltpu.VMEM((CH,), jnp.float32),
        pltpu.VMEM_SHARED((NRP,), jnp.float32),
    ],
    compiler_params=pltpu.CompilerParams(
        use_tc_tiling_on_sc=False, needs_layout_passes=False),
)


def _agg_body(z_hbm, iA_hbm, iB_hbm, dst_hbm, cidx_hbm, invcnt_hbm,
              accA_hbm, accB_hbm,
              iA_v, iB_v, dst_v, cidx_v, rowsA_v, rowsB_v, rowsC_v, s_v,
              accA_sh, accB_sh, sem):
  c = lax.axis_index("c")
  s = lax.axis_index("s")
  wid = c * NS + s

  def zba(i, _):
    rowsA_v[i // (HA // 16), pl.ds((i % (HA // 16)) * 16, 16)] = jnp.zeros(
        (16,), jnp.float32)
    return 0
  lax.fori_loop(0, CH * (HA // 16), zba, 0)

  def zbb(i, _):
    rowsC_v[i // (HB // 16), pl.ds((i % (HB // 16)) * 16, 16)] = jnp.zeros(
        (16,), jnp.float32)
    return 0
  lax.fori_loop(0, CH * (HB // 16), zbb, 0)
  for k in range(NPS // CH):
    pltpu.sync_copy(rowsA_v, accA_sh.at[pl.ds(s * NPS + k * CH, CH)])
    pltpu.sync_copy(rowsC_v, accB_sh.at[pl.ds(s * NPS + k * CH, CH)])

  plsc.subcore_barrier()

  def body(sc, _):
    pltpu.sync_copy(iA_hbm.at[wid, pl.ds(sc * SCI, SCI)], iA_v)
    pltpu.sync_copy(iB_hbm.at[wid, pl.ds(sc * SCI, SCI)], iB_v)
    pltpu.sync_copy(dst_hbm.at[wid, pl.ds(sc * SCI, SCI)], dst_v)
    pltpu.sync_copy(cidx_hbm.at[wid, pl.ds(sc * SCI, SCI)], cidx_v)
    for k in range(SCI):
      pltpu.async_copy(z_hbm.at[iA_v.at[k]], rowsA_v, sem).wait()
      pltpu.async_copy(z_hbm.at[iB_v.at[k]], rowsB_v, sem).wait()
      pltpu.async_copy(invcnt_hbm.at[cidx_v.at[k]], s_v, sem).wait()

      def sb(i, _):
        sv = plsc.load_gather(s_v, [jnp.full((16,), i, jnp.int32)])
        for q in range(HA // 16):
          rowsA_v[i, pl.ds(q * 16, 16)] = rowsA_v[i, pl.ds(q * 16, 16)] * sv
        for q in range(HB // 16):
          rowsC_v[i, pl.ds(q * 16, 16)] = rowsB_v[i, pl.ds(q * 16, 16)] * sv
        return 0
      lax.fori_loop(0, CH, sb, 0)

      pltpu.sync_copy(rowsA_v, accA_sh.at[dst_v.at[k]], add=True)
      pltpu.sync_copy(rowsC_v, accB_sh.at[dst_v.at[k]], add=True)
    return 0
  lax.fori_loop(0, NCH // SCI, body, 0)

  plsc.subcore_barrier()

  pltpu.sync_copy(accA_sh.at[pl.ds(s * NPS, NPS)],
                  accA_hbm.at[c, pl.ds(s * NPS, NPS)])
  pltpu.sync_copy(accB_sh.at[pl.ds(s * NPS, NPS)],
                  accB_hbm.at[c, pl.ds(s * NPS, NPS)])


_agg_kernel = pl.kernel(
    _agg_body,
    out_type=(jax.ShapeDtypeStruct((NC, NAP, HA), jnp.float32),
              jax.ShapeDtypeStruct((NC, NAP, HB), jnp.float32)),
    mesh=plsc.VectorSubcoreMesh(
        core_axis_name="c", subcore_axis_name="s", num_cores=NC,
        num_subcores=NS),
    scratch_types=[
        pltpu.VMEM((SCI, CH), jnp.int32),
        pltpu.VMEM((SCI, CH), jnp.int32),
        pltpu.VMEM((SCI, CH), jnp.int32),
        pltpu.VMEM((SCI, CH), jnp.int32),
        pltpu.VMEM((CH, HA), jnp.float32),
        pltpu.VMEM((CH, HA), jnp.float32),
        pltpu.VMEM((CH, HB), jnp.float32),
        pltpu.VMEM((CH,), jnp.float32),
        pltpu.VMEM_SHARED((NAP, HA), jnp.float32),
        pltpu.VMEM_SHARED((NAP, HB), jnp.float32),
        pltpu.SemaphoreType.DMA,
    ],
    compiler_params=pltpu.CompilerParams(
        use_tc_tiling_on_sc=False, needs_layout_passes=False),
)


def _z_body(x_ref, w_ref, zt_ref):
  xb = x_ref[...]
  ys = [jnp.dot(xb[:, b * BD:(b + 1) * BD], w_ref[0, b],
                preferred_element_type=jnp.float32) for b in range(NB)]
  ya = jnp.concatenate(ys[:4], axis=1)
  zt_ref[:, 0:8, :] = ya.reshape(TN // 8, 8, HA)
  zt_ref[:, 8:16, 0:HB] = ys[4].reshape(TN // 8, 8, HB)


def _z_call(x, w):
  return pl.pallas_call(
      _z_body,
      grid=(NT, R),
      in_specs=[
          pl.BlockSpec((TN, H), lambda nt, r: (nt, 0)),
          pl.BlockSpec((1, NB, BD, BD), lambda nt, r: (r, 0, 0, 0)),
      ],
      out_specs=pl.BlockSpec((TN // 8, 16, HA),
                             lambda nt, r: (r * NT + nt, 0, 0)),
      out_shape=jax.ShapeDtypeStruct((ZG, 16, HA), jnp.float32),
  )(x, w)


def _fin_body(aA_ref, aB_ref, x_ref, root_ref, bias_ref, o_ref, *, relu):
  agg = jnp.concatenate([aA_ref[0] + aA_ref[1], aB_ref[0] + aB_ref[1]],
                        axis=1)
  t = (agg
       + jnp.dot(x_ref[...], root_ref[...],
                 preferred_element_type=jnp.float32)
       + bias_ref[...])
  o_ref[...] = jnp.maximum(t, 0.0) if relu else t


def _fin_call(accA, accB, x, root, bias, relu):
  return pl.pallas_call(
      functools.partial(_fin_body, relu=relu),
      grid=(NT,),
      in_specs=[
          pl.BlockSpec((NC, TN, HA), lambda nt: (0, nt, 0)),
          pl.BlockSpec((NC, TN, HB), lambda nt: (0, nt, 0)),
          pl.BlockSpec((TN, H), lambda nt: (nt, 0)),
          pl.BlockSpec((H, H), lambda nt: (0, 0)),
          pl.BlockSpec((1, H), lambda nt: (0, 0)),
      ],
      out_specs=pl.BlockSpec((TN, H), lambda nt: (nt, 0)),
      out_shape=jax.ShapeDtypeStruct((N, H), jnp.float32),
  )(accA, accB, x, root, bias)


def kernel(node_emb, edge_index, edge_type, W1, root1, bias1, W2, root2,
           bias2):
  src = edge_index[0].astype(jnp.int32)
  dst = edge_index[1].astype(jnp.int32)
  et = edge_type.astype(jnp.int32)

  g = et * N + src
  iA = (g // 8) * 16 + (g % 8)
  iB = iA + 8
  iA_r = iA.reshape(NW, NCH, CH)
  iB_r = iB.reshape(NW, NCH, CH)
  dstr = dst.reshape(NW, NCH, CH)
  cidx_flat = dst * R + et
  cidx32 = cidx_flat.reshape(NW, NCH, CH)
  cidx16 = cidx_flat.reshape(NS, CNCH, CH)
  bias1_2d = bias1.reshape(1, H)
  bias2_2d = bias2.reshape(1, H)

  invcnt = _cnt_kernel(cidx16)

  def layer(x, w, root, bias, relu):
    zt = _z_call(x, w)
    zv = zt.reshape(ZG * 16, HA)
    accA, accB = _agg_kernel(zv, iA_r, iB_r, dstr, cidx32, invcnt)
    return _fin_call(accA, accB, x, root, bias, relu)

  h1 = layer(node_emb, W1, root1, bias1_2d, relu=True)
  return layer(h1, W2, root2, bias2_2d, relu=False)


# speedup vs baseline: 67.0017x; 1.2813x over previous
import functools

import jax
import jax.numpy as jnp
from jax import lax
from jax.experimental import pallas as pl
from jax.experimental.pallas import tpu as pltpu
from jax.experimental.pallas import tpu_sc as plsc

N = 10000
H = 160
HA = 128
HB = 32
R = 50
NB = 5
BD = 32
E = 320000

NC = 2
NS = 16
NW = NC * NS

EPW = E // NW
EPAD = 32
CH = 48
NCH = (EPW + EPAD) // CH
SCI = 11

CCH = 80
CNCH = E // (NS * CCH)

NRP = 512000
CSPAN = NRP // NS
CZCH = 4000

NAP = 10240
NPS = NAP // NS

NT = 10
TN = N // NT
ZG = R * N // 8


def _cnt_body(cidx_hbm, invcnt_hbm, cidx_v, buf_v, ones_v, cnt_sh):
  c = lax.axis_index("c")
  s = lax.axis_index("s")

  @pl.when(c == 0)
  def _zero():
    def zb(i, _):
      buf_v[pl.ds(i * 16, 16)] = jnp.zeros((16,), jnp.float32)
      return 0
    lax.fori_loop(0, CZCH // 16, zb, 0)
    base = s * CSPAN
    for k in range(CSPAN // CZCH):
      pltpu.sync_copy(buf_v, cnt_sh.at[pl.ds(base + k * CZCH, CZCH)])

  plsc.subcore_barrier()

  @pl.when(c == 0)
  def _count():
    for q in range(CCH // 16):
      ones_v[pl.ds(q * 16, 16)] = jnp.ones((16,), jnp.float32)
    pltpu.sync_copy(cidx_hbm.at[s], cidx_v)

    def body(j, _):
      pltpu.sync_copy(ones_v, cnt_sh.at[cidx_v.at[j]], add=True)
      return 0
    lax.fori_loop(0, CNCH, body, 0)

  plsc.subcore_barrier()

  @pl.when(c == 0)
  def _inv():
    base = s * CSPAN
    for k in range(CSPAN // CZCH):
      pltpu.sync_copy(cnt_sh.at[pl.ds(base + k * CZCH, CZCH)], buf_v)

      def ib(i, _):
        v = buf_v[pl.ds(i * 16, 16)]
        buf_v[pl.ds(i * 16, 16)] = 1.0 / jnp.maximum(v, 1.0)
        return 0
      lax.fori_loop(0, CZCH // 16, ib, 0)
      pltpu.sync_copy(buf_v, invcnt_hbm.at[pl.ds(base + k * CZCH, CZCH)])


_cnt_kernel = pl.kernel(
    _cnt_body,
    out_type=jax.ShapeDtypeStruct((NRP,), jnp.float32),
    mesh=plsc.VectorSubcoreMesh(
        core_axis_name="c", subcore_axis_name="s", num_cores=NC,
        num_subcores=NS),
    scratch_types=[
        pltpu.VMEM((CNCH, CCH), jnp.int32),
        pltpu.VMEM((CZCH,), jnp.float32),
        pltpu.VMEM((CCH,), jnp.float32),
        pltpu.VMEM_SHARED((NRP,), jnp.float32),
    ],
    compiler_params=pltpu.CompilerParams(
        use_tc_tiling_on_sc=False, needs_layout_passes=False),
)


def _agg_body(z_hbm, iA_hbm, iB_hbm, dst_hbm, cidx_hbm, invcnt_hbm,
              accA_hbm, accB_hbm,
              iA_v, iB_v, dst_v, cidx_v, rA0, rA1, rB0, rB1, rC_v, s0, s1,
              accA_sh, accB_sh, semg0, semg1):
  c = lax.axis_index("c")
  s = lax.axis_index("s")
  wid = c * NS + s

  def zba(i, _):
    rA0[i // (HA // 16), pl.ds((i % (HA // 16)) * 16, 16)] = jnp.zeros(
        (16,), jnp.float32)
    return 0
  lax.fori_loop(0, CH * (HA // 16), zba, 0)

  def zbb(i, _):
    rC_v[i // (HB // 16), pl.ds((i % (HB // 16)) * 16, 16)] = jnp.zeros(
        (16,), jnp.float32)
    return 0
  lax.fori_loop(0, CH * (HB // 16), zbb, 0)
  for k in range(NPS // 32):
    pltpu.sync_copy(rA0.at[pl.ds(0, 32)],
                    accA_sh.at[pl.ds(s * NPS + k * 32, 32)])
    pltpu.sync_copy(rC_v.at[pl.ds(0, 32)],
                    accB_sh.at[pl.ds(s * NPS + k * 32, 32)])

  plsc.subcore_barrier()

  rA = (rA0, rA1)
  rB = (rB0, rB1)
  sv_ = (s0, s1)
  sg = (semg0, semg1)

  def body(sc, _):
    pltpu.sync_copy(iA_hbm.at[wid, pl.ds(sc * SCI, SCI)], iA_v)
    pltpu.sync_copy(iB_hbm.at[wid, pl.ds(sc * SCI, SCI)], iB_v)
    pltpu.sync_copy(dst_hbm.at[wid, pl.ds(sc * SCI, SCI)], dst_v)
    pltpu.sync_copy(cidx_hbm.at[wid, pl.ds(sc * SCI, SCI)], cidx_v)

    g = [[None] * 3, [None] * 3]

    def issue(k, p):
      g[p][0] = pltpu.async_copy(z_hbm.at[iA_v.at[k]], rA[p], sg[p])
      g[p][1] = pltpu.async_copy(z_hbm.at[iB_v.at[k]], rB[p], sg[p])
      g[p][2] = pltpu.async_copy(invcnt_hbm.at[cidx_v.at[k]], sv_[p], sg[p])

    issue(0, 0)
    for k in range(SCI):
      p = k % 2
      if k + 1 < SCI:
        issue(k + 1, 1 - p)
      for d in g[p]:
        d.wait()

      def sb(i, _):
        sv = plsc.load_gather(sv_[p], [jnp.full((16,), i, jnp.int32)])
        for q in range(HA // 16):
          rA[p][i, pl.ds(q * 16, 16)] = rA[p][i, pl.ds(q * 16, 16)] * sv
        for q in range(HB // 16):
          rC_v[i, pl.ds(q * 16, 16)] = rB[p][i, pl.ds(q * 16, 16)] * sv
        return 0
      lax.fori_loop(0, CH, sb, 0)

      pltpu.sync_copy(rA[p], accA_sh.at[dst_v.at[k]], add=True)
      pltpu.sync_copy(rC_v, accB_sh.at[dst_v.at[k]], add=True)
    return 0
  lax.fori_loop(0, NCH // SCI, body, 0)

  plsc.subcore_barrier()

  pltpu.sync_copy(accA_sh.at[pl.ds(s * NPS, NPS)],
                  accA_hbm.at[c, pl.ds(s * NPS, NPS)])
  pltpu.sync_copy(accB_sh.at[pl.ds(s * NPS, NPS)],
                  accB_hbm.at[c, pl.ds(s * NPS, NPS)])


_agg_kernel = pl.kernel(
    _agg_body,
    out_type=(jax.ShapeDtypeStruct((NC, NAP, HA), jnp.float32),
              jax.ShapeDtypeStruct((NC, NAP, HB), jnp.float32)),
    mesh=plsc.VectorSubcoreMesh(
        core_axis_name="c", subcore_axis_name="s", num_cores=NC,
        num_subcores=NS),
    scratch_types=[
        pltpu.VMEM((SCI, CH), jnp.int32),
        pltpu.VMEM((SCI, CH), jnp.int32),
        pltpu.VMEM((SCI, CH), jnp.int32),
        pltpu.VMEM((SCI, CH), jnp.int32),
        pltpu.VMEM((CH, HA), jnp.float32),
        pltpu.VMEM((CH, HA), jnp.float32),
        pltpu.VMEM((CH, HA), jnp.float32),
        pltpu.VMEM((CH, HA), jnp.float32),
        pltpu.VMEM((CH, HB), jnp.float32),
        pltpu.VMEM((CH,), jnp.float32),
        pltpu.VMEM((CH,), jnp.float32),
        pltpu.VMEM_SHARED((NAP, HA), jnp.float32),
        pltpu.VMEM_SHARED((NAP, HB), jnp.float32),
        pltpu.SemaphoreType.DMA,
        pltpu.SemaphoreType.DMA,
    ],
    compiler_params=pltpu.CompilerParams(
        use_tc_tiling_on_sc=False, needs_layout_passes=False),
)


def _z_body(x_ref, w_ref, zt_ref):
  xb = x_ref[...]
  ys = [jnp.dot(xb[:, b * BD:(b + 1) * BD], w_ref[0, b],
                preferred_element_type=jnp.float32) for b in range(NB)]
  ya = jnp.concatenate(ys[:4], axis=1)
  zt_ref[:, 0:8, :] = ya.reshape(TN // 8, 8, HA)
  zt_ref[:, 8:16, 0:HB] = ys[4].reshape(TN // 8, 8, HB)


def _z_call(x, w):
  return pl.pallas_call(
      _z_body,
      grid=(NT, R),
      in_specs=[
          pl.BlockSpec((TN, H), lambda nt, r: (nt, 0)),
          pl.BlockSpec((1, NB, BD, BD), lambda nt, r: (r, 0, 0, 0)),
      ],
      out_specs=pl.BlockSpec((TN // 8, 16, HA),
                             lambda nt, r: (r * NT + nt, 0, 0)),
      out_shape=jax.ShapeDtypeStruct((ZG, 16, HA), jnp.float32),
  )(x, w)


def _fin_body(aA_ref, aB_ref, x_ref, root_ref, bias_ref, o_ref, *, relu):
  agg = jnp.concatenate([aA_ref[0] + aA_ref[1], aB_ref[0] + aB_ref[1]],
                        axis=1)
  t = (agg
       + jnp.dot(x_ref[...], root_ref[...],
                 preferred_element_type=jnp.float32)
       + bias_ref[...])
  o_ref[...] = jnp.maximum(t, 0.0) if relu else t


def _fin_call(accA, accB, x, root, bias, relu):
  return pl.pallas_call(
      functools.partial(_fin_body, relu=relu),
      grid=(NT,),
      in_specs=[
          pl.BlockSpec((NC, TN, HA), lambda nt: (0, nt, 0)),
          pl.BlockSpec((NC, TN, HB), lambda nt: (0, nt, 0)),
          pl.BlockSpec((TN, H), lambda nt: (nt, 0)),
          pl.BlockSpec((H, H), lambda nt: (0, 0)),
          pl.BlockSpec((1, H), lambda nt: (0, 0)),
      ],
      out_specs=pl.BlockSpec((TN, H), lambda nt: (nt, 0)),
      out_shape=jax.ShapeDtypeStruct((N, H), jnp.float32),
  )(accA, accB, x, root, bias)


def kernel(node_emb, edge_index, edge_type, W1, root1, bias1, W2, root2,
           bias2):
  src = edge_index[0].astype(jnp.int32)
  dst = edge_index[1].astype(jnp.int32)
  et = edge_type.astype(jnp.int32)

  g = et * N + src
  iA = (g // 8) * 16 + (g % 8)
  iB = iA + 8
  cidx_flat = dst * R + et

  def wpad(a, padvals):
    pad = jnp.broadcast_to(padvals, (NW, EPAD)).astype(jnp.int32)
    return jnp.concatenate([a.reshape(NW, EPW), pad], axis=1).reshape(
        NW, NCH, CH)

  trash = N + 8 + jnp.arange(EPAD, dtype=jnp.int32)
  iA_r = wpad(iA, jnp.int32(0))
  iB_r = wpad(iB, jnp.int32(8))
  dstr = wpad(dst, trash)
  cidx32 = wpad(cidx_flat, jnp.int32(NRP - 8))
  cidx16 = cidx_flat.reshape(NS, CNCH, CCH)
  bias1_2d = bias1.reshape(1, H)
  bias2_2d = bias2.reshape(1, H)

  invcnt = _cnt_kernel(cidx16)

  def layer(x, w, root, bias, relu):
    zt = _z_call(x, w)
    zv = zt.reshape(ZG * 16, HA)
    accA, accB = _agg_kernel(zv, iA_r, iB_r, dstr, cidx32, invcnt)
    return _fin_call(accA, accB, x, root, bias, relu)

  h1 = layer(node_emb, W1, root1, bias1_2d, relu=True)
  return layer(h1, W2, root2, bias2_2d, relu=False)


# speedup vs baseline: 69.1423x; 1.0319x over previous
import functools

import jax
import jax.numpy as jnp
from jax import lax
from jax.experimental import pallas as pl
from jax.experimental.pallas import tpu as pltpu
from jax.experimental.pallas import tpu_sc as plsc

N = 10000
H = 160
HA = 128
HB = 32
R = 50
NB = 5
BD = 32
E = 320000

NC = 2
NS = 16
NW = NC * NS

EPW = E // NW
EPAD = 32
CH = 48
NCH = (EPW + EPAD) // CH
SCI = 11

CCH = 80
CNCH = E // (NS * CCH)

NRP = 512000
CSPAN = NRP // NS
CZCH = 4000

NAP = 10240
NPS = NAP // NS

NT = 10
TN = N // NT
ZG = R * N // 8


def _cnt_body(cidx_hbm, invcnt_hbm, cidx_v, buf_v, ones_v, cnt_sh):
  c = lax.axis_index("c")
  s = lax.axis_index("s")

  @pl.when(c == 0)
  def _zero():
    def zb(i, _):
      buf_v[pl.ds(i * 16, 16)] = jnp.zeros((16,), jnp.float32)
      return 0
    lax.fori_loop(0, CZCH // 16, zb, 0)
    base = s * CSPAN
    for k in range(CSPAN // CZCH):
      pltpu.sync_copy(buf_v, cnt_sh.at[pl.ds(base + k * CZCH, CZCH)])

  plsc.subcore_barrier()

  @pl.when(c == 0)
  def _count():
    for q in range(CCH // 16):
      ones_v[pl.ds(q * 16, 16)] = jnp.ones((16,), jnp.float32)
    pltpu.sync_copy(cidx_hbm.at[s], cidx_v)

    def body(j, _):
      pltpu.sync_copy(ones_v, cnt_sh.at[cidx_v.at[j]], add=True)
      return 0
    lax.fori_loop(0, CNCH, body, 0)

  plsc.subcore_barrier()

  @pl.when(c == 0)
  def _inv():
    base = s * CSPAN
    for k in range(CSPAN // CZCH):
      pltpu.sync_copy(cnt_sh.at[pl.ds(base + k * CZCH, CZCH)], buf_v)

      def ib(i, _):
        v = buf_v[pl.ds(i * 16, 16)]
        buf_v[pl.ds(i * 16, 16)] = 1.0 / jnp.maximum(v, 1.0)
        return 0
      lax.fori_loop(0, CZCH // 16, ib, 0)
      pltpu.sync_copy(buf_v, invcnt_hbm.at[pl.ds(base + k * CZCH, CZCH)])


_cnt_kernel = pl.kernel(
    _cnt_body,
    out_type=jax.ShapeDtypeStruct((NRP,), jnp.float32),
    mesh=plsc.VectorSubcoreMesh(
        core_axis_name="c", subcore_axis_name="s", num_cores=NC,
        num_subcores=NS),
    scratch_types=[
        pltpu.VMEM((CNCH, CCH), jnp.int32),
        pltpu.VMEM((CZCH,), jnp.float32),
        pltpu.VMEM((CCH,), jnp.float32),
        pltpu.VMEM_SHARED((NRP,), jnp.float32),
    ],
    compiler_params=pltpu.CompilerParams(
        use_tc_tiling_on_sc=False, needs_layout_passes=False),
)


def _agg_body(z_hbm, iA_hbm, iB_hbm, dst_hbm, cidx_hbm, invcnt_hbm,
              accA_hbm, accB_hbm,
              iA_v, iB_v, dst_v, cidx_v, rA0, rA1, rB0, rB1, rC_v, s0, s1,
              accA_sh, accB_sh, semg0, semg1):
  c = lax.axis_index("c")
  s = lax.axis_index("s")
  wid = c * NS + s

  def zba(i, _):
    rA0[i // (HA // 16), pl.ds((i % (HA // 16)) * 16, 16)] = jnp.zeros(
        (16,), jnp.float32)
    return 0
  lax.fori_loop(0, CH * (HA // 16), zba, 0)

  def zbb(i, _):
    rC_v[i // (HB // 16), pl.ds((i % (HB // 16)) * 16, 16)] = jnp.zeros(
        (16,), jnp.float32)
    return 0
  lax.fori_loop(0, CH * (HB // 16), zbb, 0)
  for k in range(NPS // 32):
    pltpu.sync_copy(rA0.at[pl.ds(0, 32)],
                    accA_sh.at[pl.ds(s * NPS + k * 32, 32)])
    pltpu.sync_copy(rC_v.at[pl.ds(0, 32)],
                    accB_sh.at[pl.ds(s * NPS + k * 32, 32)])

  plsc.subcore_barrier()

  rA = (rA0, rA1)
  rB = (rB0, rB1)
  sv_ = (s0, s1)
  sg = (semg0, semg1)

  def body(sc, _):
    pltpu.sync_copy(iA_hbm.at[wid, pl.ds(sc * SCI, SCI)], iA_v)
    pltpu.sync_copy(iB_hbm.at[wid, pl.ds(sc * SCI, SCI)], iB_v)
    pltpu.sync_copy(dst_hbm.at[wid, pl.ds(sc * SCI, SCI)], dst_v)
    pltpu.sync_copy(cidx_hbm.at[wid, pl.ds(sc * SCI, SCI)], cidx_v)

    g = [[None] * 3, [None] * 3]

    def issue(k, p):
      g[p][0] = pltpu.async_copy(z_hbm.at[iA_v.at[k]], rA[p], sg[p])
      g[p][1] = pltpu.async_copy(z_hbm.at[iB_v.at[k]], rB[p], sg[p])
      g[p][2] = pltpu.async_copy(invcnt_hbm.at[cidx_v.at[k]], sv_[p], sg[p])

    issue(0, 0)
    for k in range(SCI):
      p = k % 2
      if k + 1 < SCI:
        issue(k + 1, 1 - p)
      for d in g[p]:
        d.wait()

      def sb(i, _):
        sv = plsc.load_gather(sv_[p], [jnp.full((16,), i, jnp.int32)])
        for q in range(HA // 16):
          rA[p][i, pl.ds(q * 16, 16)] = rA[p][i, pl.ds(q * 16, 16)] * sv
        for q in range(HB // 16):
          rC_v[i, pl.ds(q * 16, 16)] = rB[p][i, pl.ds(q * 16, 16)] * sv
        return 0
      lax.fori_loop(0, CH, sb, 0)

      pltpu.sync_copy(rA[p], accA_sh.at[dst_v.at[k]], add=True)
      pltpu.sync_copy(rC_v, accB_sh.at[dst_v.at[k]], add=True)
    return 0
  lax.fori_loop(0, NCH // SCI, body, 0)

  plsc.subcore_barrier()

  pltpu.sync_copy(accA_sh.at[pl.ds(s * NPS, NPS)],
                  accA_hbm.at[c, pl.ds(s * NPS, NPS)])
  pltpu.sync_copy(accB_sh.at[pl.ds(s * NPS, NPS)],
                  accB_hbm.at[c, pl.ds(s * NPS, NPS)])


_agg_kernel = pl.kernel(
    _agg_body,
    out_type=(jax.ShapeDtypeStruct((NC, NAP, HA), jnp.float32),
              jax.ShapeDtypeStruct((NC, NAP, HB), jnp.float32)),
    mesh=plsc.VectorSubcoreMesh(
        core_axis_name="c", subcore_axis_name="s", num_cores=NC,
        num_subcores=NS),
    scratch_types=[
        pltpu.VMEM((SCI, CH), jnp.int32),
        pltpu.VMEM((SCI, CH), jnp.int32),
        pltpu.VMEM((SCI, CH), jnp.int32),
        pltpu.VMEM((SCI, CH), jnp.int32),
        pltpu.VMEM((CH, HA), jnp.float32),
        pltpu.VMEM((CH, HA), jnp.float32),
        pltpu.VMEM((CH, HA), jnp.float32),
        pltpu.VMEM((CH, HA), jnp.float32),
        pltpu.VMEM((CH, HB), jnp.float32),
        pltpu.VMEM((CH,), jnp.float32),
        pltpu.VMEM((CH,), jnp.float32),
        pltpu.VMEM_SHARED((NAP, HA), jnp.float32),
        pltpu.VMEM_SHARED((NAP, HB), jnp.float32),
        pltpu.SemaphoreType.DMA,
        pltpu.SemaphoreType.DMA,
    ],
    compiler_params=pltpu.CompilerParams(
        use_tc_tiling_on_sc=False, needs_layout_passes=False),
)


def _z_body(x_ref, w_ref, zt_ref):
  y = jnp.dot(x_ref[...], w_ref[0], preferred_element_type=jnp.float32)
  zt_ref[:, 0:8, :] = y[:, 0:HA].reshape(TN // 8, 8, HA)
  zt_ref[:, 8:16, 0:HB] = y[:, HA:H].reshape(TN // 8, 8, HB)


def _z_call(x, wd):
  return pl.pallas_call(
      _z_body,
      grid=(NT, R),
      in_specs=[
          pl.BlockSpec((TN, H), lambda nt, r: (nt, 0)),
          pl.BlockSpec((1, H, H), lambda nt, r: (r, 0, 0)),
      ],
      out_specs=pl.BlockSpec((TN // 8, 16, HA),
                             lambda nt, r: (r * NT + nt, 0, 0)),
      out_shape=jax.ShapeDtypeStruct((ZG, 16, HA), jnp.float32),
  )(x, wd)


def _fin_body(aA_ref, aB_ref, x_ref, root_ref, bias_ref, o_ref, *, relu):
  agg = jnp.concatenate([aA_ref[0] + aA_ref[1], aB_ref[0] + aB_ref[1]],
                        axis=1)
  t = (agg
       + jnp.dot(x_ref[...], root_ref[...],
                 preferred_element_type=jnp.float32)
       + bias_ref[...])
  o_ref[...] = jnp.maximum(t, 0.0) if relu else t


def _fin_call(accA, accB, x, root, bias, relu):
  return pl.pallas_call(
      functools.partial(_fin_body, relu=relu),
      grid=(NT,),
      in_specs=[
          pl.BlockSpec((NC, TN, HA), lambda nt: (0, nt, 0)),
          pl.BlockSpec((NC, TN, HB), lambda nt: (0, nt, 0)),
          pl.BlockSpec((TN, H), lambda nt: (nt, 0)),
          pl.BlockSpec((H, H), lambda nt: (0, 0)),
          pl.BlockSpec((1, H), lambda nt: (0, 0)),
      ],
      out_specs=pl.BlockSpec((TN, H), lambda nt: (nt, 0)),
      out_shape=jax.ShapeDtypeStruct((N, H), jnp.float32),
  )(accA, accB, x, root, bias)


def kernel(node_emb, edge_index, edge_type, W1, root1, bias1, W2, root2,
           bias2):
  src = edge_index[0].astype(jnp.int32)
  dst = edge_index[1].astype(jnp.int32)
  et = edge_type.astype(jnp.int32)

  g = et * N + src
  iA = (g // 8) * 16 + (g % 8)
  iB = iA + 8
  cidx_flat = dst * R + et

  def wpad(a, padvals):
    pad = jnp.broadcast_to(padvals, (NW, EPAD)).astype(jnp.int32)
    return jnp.concatenate([a.reshape(NW, EPW), pad], axis=1).reshape(
        NW, NCH, CH)

  trash = N + 8 + jnp.arange(EPAD, dtype=jnp.int32)
  iA_r = wpad(iA, jnp.int32(0))
  iB_r = wpad(iB, jnp.int32(8))
  dstr = wpad(dst, trash)
  cidx32 = wpad(cidx_flat, jnp.int32(NRP - 8))
  cidx16 = cidx_flat.reshape(NS, CNCH, CCH)
  bias1_2d = bias1.reshape(1, H)
  bias2_2d = bias2.reshape(1, H)

  invcnt = _cnt_kernel(cidx16)

  def blockdiag(w):
    wd = jnp.zeros((R, H, H), jnp.float32)
    for b in range(NB):
      wd = wd.at[:, b * BD:(b + 1) * BD, b * BD:(b + 1) * BD].set(w[:, b])
    return wd

  Wd1 = blockdiag(W1)
  Wd2 = blockdiag(W2)

  def layer(x, w, root, bias, relu):
    zt = _z_call(x, w)
    zv = zt.reshape(ZG * 16, HA)
    accA, accB = _agg_kernel(zv, iA_r, iB_r, dstr, cidx32, invcnt)
    return _fin_call(accA, accB, x, root, bias, relu)

  h1 = layer(node_emb, Wd1, root1, bias1_2d, relu=True)
  return layer(h1, Wd2, root2, bias2_2d, relu=False)


# speedup vs baseline: 82.1679x; 1.1884x over previous
import functools

import jax
import jax.numpy as jnp
from jax import lax
from jax.experimental import pallas as pl
from jax.experimental.pallas import tpu as pltpu
from jax.experimental.pallas import tpu_sc as plsc

N = 10000
H = 160
HA = 128
HB = 32
R = 50
NB = 5
BD = 32
E = 320000

NC = 2
NS = 16
NW = NC * NS

EPW = E // NW
EPAD = 32
CH = 48
NCH = (EPW + EPAD) // CH
SCI = 11

CCH = 80
CNCH = E // (NS * CCH)

NRP = 512000
CSPAN = NRP // NS
CZCH = 4000

NAP = 10240
NPS = NAP // NS

NT = 5
TN = N // NT
NTF = 10
TF = N // NTF
ZG = R * N // 8


def _cnt_body(cidx_hbm, invcnt_hbm, cidx_v, buf_v, ones_v, cnt_sh):
  c = lax.axis_index("c")
  s = lax.axis_index("s")

  @pl.when(c == 0)
  def _zero():
    def zb(i, _):
      buf_v[pl.ds(i * 16, 16)] = jnp.zeros((16,), jnp.float32)
      return 0
    lax.fori_loop(0, CZCH // 16, zb, 0)
    base = s * CSPAN
    for k in range(CSPAN // CZCH):
      pltpu.sync_copy(buf_v, cnt_sh.at[pl.ds(base + k * CZCH, CZCH)])

  plsc.subcore_barrier()

  @pl.when(c == 0)
  def _count():
    for q in range(CCH // 16):
      ones_v[pl.ds(q * 16, 16)] = jnp.ones((16,), jnp.float32)
    pltpu.sync_copy(cidx_hbm.at[s], cidx_v)

    def body(j, _):
      pltpu.sync_copy(ones_v, cnt_sh.at[cidx_v.at[j]], add=True)
      return 0
    lax.fori_loop(0, CNCH, body, 0)

  plsc.subcore_barrier()

  @pl.when(c == 0)
  def _inv():
    base = s * CSPAN
    for k in range(CSPAN // CZCH):
      pltpu.sync_copy(cnt_sh.at[pl.ds(base + k * CZCH, CZCH)], buf_v)

      def ib(i, _):
        v = buf_v[pl.ds(i * 16, 16)]
        buf_v[pl.ds(i * 16, 16)] = 1.0 / jnp.maximum(v, 1.0)
        return 0
      lax.fori_loop(0, CZCH // 16, ib, 0)
      pltpu.sync_copy(buf_v, invcnt_hbm.at[pl.ds(base + k * CZCH, CZCH)])


_cnt_kernel = pl.kernel(
    _cnt_body,
    out_type=jax.ShapeDtypeStruct((NRP,), jnp.float32),
    mesh=plsc.VectorSubcoreMesh(
        core_axis_name="c", subcore_axis_name="s", num_cores=NC,
        num_subcores=NS),
    scratch_types=[
        pltpu.VMEM((CNCH, CCH), jnp.int32),
        pltpu.VMEM((CZCH,), jnp.float32),
        pltpu.VMEM((CCH,), jnp.float32),
        pltpu.VMEM_SHARED((NRP,), jnp.float32),
    ],
    compiler_params=pltpu.CompilerParams(
        use_tc_tiling_on_sc=False, needs_layout_passes=False),
)


def _agg_body(z_hbm, iA_hbm, iB_hbm, dst_hbm, cidx_hbm, invcnt_hbm,
              accA_hbm, accB_hbm,
              iA_v, iB_v, dst_v, cidx_v, rA0, rA1, rB0, rB1, rC_v, s0, s1,
              accA_sh, accB_sh, semg0, semg1):
  c = lax.axis_index("c")
  s = lax.axis_index("s")
  wid = c * NS + s

  def zba(i, _):
    rA0[i // (HA // 16), pl.ds((i % (HA // 16)) * 16, 16)] = jnp.zeros(
        (16,), jnp.float32)
    return 0
  lax.fori_loop(0, CH * (HA // 16), zba, 0)

  def zbb(i, _):
    rC_v[i // (HB // 16), pl.ds((i % (HB // 16)) * 16, 16)] = jnp.zeros(
        (16,), jnp.float32)
    return 0
  lax.fori_loop(0, CH * (HB // 16), zbb, 0)
  for k in range(NPS // 32):
    pltpu.sync_copy(rA0.at[pl.ds(0, 32)],
                    accA_sh.at[pl.ds(s * NPS + k * 32, 32)])
    pltpu.sync_copy(rC_v.at[pl.ds(0, 32)],
                    accB_sh.at[pl.ds(s * NPS + k * 32, 32)])

  plsc.subcore_barrier()

  rA = (rA0, rA1)
  rB = (rB0, rB1)
  sv_ = (s0, s1)
  sg = (semg0, semg1)

  def body(sc, _):
    pltpu.sync_copy(iA_hbm.at[wid, pl.ds(sc * SCI, SCI)], iA_v)
    pltpu.sync_copy(iB_hbm.at[wid, pl.ds(sc * SCI, SCI)], iB_v)
    pltpu.sync_copy(dst_hbm.at[wid, pl.ds(sc * SCI, SCI)], dst_v)
    pltpu.sync_copy(cidx_hbm.at[wid, pl.ds(sc * SCI, SCI)], cidx_v)

    g = [[None] * 3, [None] * 3]

    def issue(k, p):
      g[p][0] = pltpu.async_copy(z_hbm.at[iA_v.at[k]], rA[p], sg[p])
      g[p][1] = pltpu.async_copy(z_hbm.at[iB_v.at[k]], rB[p], sg[p])
      g[p][2] = pltpu.async_copy(invcnt_hbm.at[cidx_v.at[k]], sv_[p], sg[p])

    issue(0, 0)
    for k in range(SCI):
      p = k % 2
      if k + 1 < SCI:
        issue(k + 1, 1 - p)
      for d in g[p]:
        d.wait()

      def sb(i, _):
        sv = plsc.load_gather(sv_[p], [jnp.full((16,), i, jnp.int32)])
        for q in range(HA // 16):
          rA[p][i, pl.ds(q * 16, 16)] = rA[p][i, pl.ds(q * 16, 16)] * sv
        for q in range(HB // 16):
          rC_v[i, pl.ds(q * 16, 16)] = rB[p][i, pl.ds(q * 16, 16)] * sv
        return 0
      lax.fori_loop(0, CH, sb, 0)

      pltpu.sync_copy(rA[p], accA_sh.at[dst_v.at[k]], add=True)
      pltpu.sync_copy(rC_v, accB_sh.at[dst_v.at[k]], add=True)
    return 0
  lax.fori_loop(0, NCH // SCI, body, 0)

  plsc.subcore_barrier()

  pltpu.sync_copy(accA_sh.at[pl.ds(s * NPS, NPS)],
                  accA_hbm.at[c, pl.ds(s * NPS, NPS)])
  pltpu.sync_copy(accB_sh.at[pl.ds(s * NPS, NPS)],
                  accB_hbm.at[c, pl.ds(s * NPS, NPS)])


_agg_kernel = pl.kernel(
    _agg_body,
    out_type=(jax.ShapeDtypeStruct((NC, NAP, HA), jnp.float32),
              jax.ShapeDtypeStruct((NC, NAP, HB), jnp.float32)),
    mesh=plsc.VectorSubcoreMesh(
        core_axis_name="c", subcore_axis_name="s", num_cores=NC,
        num_subcores=NS),
    scratch_types=[
        pltpu.VMEM((SCI, CH), jnp.int32),
        pltpu.VMEM((SCI, CH), jnp.int32),
        pltpu.VMEM((SCI, CH), jnp.int32),
        pltpu.VMEM((SCI, CH), jnp.int32),
        pltpu.VMEM((CH, HA), jnp.float32),
        pltpu.VMEM((CH, HA), jnp.float32),
        pltpu.VMEM((CH, HA), jnp.float32),
        pltpu.VMEM((CH, HA), jnp.float32),
        pltpu.VMEM((CH, HB), jnp.float32),
        pltpu.VMEM((CH,), jnp.float32),
        pltpu.VMEM((CH,), jnp.float32),
        pltpu.VMEM_SHARED((NAP, HA), jnp.float32),
        pltpu.VMEM_SHARED((NAP, HB), jnp.float32),
        pltpu.SemaphoreType.DMA,
        pltpu.SemaphoreType.DMA,
    ],
    compiler_params=pltpu.CompilerParams(
        use_tc_tiling_on_sc=False, needs_layout_passes=False),
)


def _z_body(x_ref, w_ref, zt_ref):
  y = jnp.dot(x_ref[...], w_ref[0], preferred_element_type=jnp.float32)
  zt_ref[:, 0:8, :] = y[:, 0:HA].reshape(TN // 8, 8, HA)
  zt_ref[:, 8:16, 0:HB] = y[:, HA:H].reshape(TN // 8, 8, HB)


def _z_call(x, wd):
  return pl.pallas_call(
      _z_body,
      grid=(NT, R),
      in_specs=[
          pl.BlockSpec((TN, H), lambda nt, r: (nt, 0)),
          pl.BlockSpec((1, H, H), lambda nt, r: (r, 0, 0)),
      ],
      out_specs=pl.BlockSpec((TN // 8, 16, HA),
                             lambda nt, r: (r * NT + nt, 0, 0)),
      out_shape=jax.ShapeDtypeStruct((ZG, 16, HA), jnp.float32),
  )(x, wd)


def _fin_body(aA_ref, aB_ref, x_ref, root_ref, bias_ref, o_ref, *, relu):
  agg = jnp.concatenate([aA_ref[0] + aA_ref[1], aB_ref[0] + aB_ref[1]],
                        axis=1)
  t = (agg
       + jnp.dot(x_ref[...], root_ref[...],
                 preferred_element_type=jnp.float32)
       + bias_ref[...])
  o_ref[...] = jnp.maximum(t, 0.0) if relu else t


def _fin_call(accA, accB, x, root, bias, relu):
  return pl.pallas_call(
      functools.partial(_fin_body, relu=relu),
      grid=(NTF,),
      in_specs=[
          pl.BlockSpec((NC, TF, HA), lambda nt: (0, nt, 0)),
          pl.BlockSpec((NC, TF, HB), lambda nt: (0, nt, 0)),
          pl.BlockSpec((TF, H), lambda nt: (nt, 0)),
          pl.BlockSpec((H, H), lambda nt: (0, 0)),
          pl.BlockSpec((1, H), lambda nt: (0, 0)),
      ],
      out_specs=pl.BlockSpec((TF, H), lambda nt: (nt, 0)),
      out_shape=jax.ShapeDtypeStruct((N, H), jnp.float32),
  )(accA, accB, x, root, bias)


def kernel(node_emb, edge_index, edge_type, W1, root1, bias1, W2, root2,
           bias2):
  src = edge_index[0].astype(jnp.int32)
  dst = edge_index[1].astype(jnp.int32)
  et = edge_type.astype(jnp.int32)

  g = et * N + src
  iA = (g // 8) * 16 + (g % 8)
  iB = iA + 8
  cidx_flat = dst * R + et

  def wpad(a, padvals):
    pad = jnp.broadcast_to(padvals, (NW, EPAD)).astype(jnp.int32)
    return jnp.concatenate([a.reshape(NW, EPW), pad], axis=1).reshape(
        NW, NCH, CH)

  trash = N + 8 + jnp.arange(EPAD, dtype=jnp.int32)
  iA_r = wpad(iA, jnp.int32(0))
  iB_r = wpad(iB, jnp.int32(8))
  dstr = wpad(dst, trash)
  cidx32 = wpad(cidx_flat, jnp.int32(NRP - 8))
  cidx16 = cidx_flat.reshape(NS, CNCH, CCH)
  bias1_2d = bias1.reshape(1, H)
  bias2_2d = bias2.reshape(1, H)

  invcnt = _cnt_kernel(cidx16)

  def blockdiag(w):
    wd = jnp.zeros((R, H, H), jnp.float32)
    for b in range(NB):
      wd = wd.at[:, b * BD:(b + 1) * BD, b * BD:(b + 1) * BD].set(w[:, b])
    return wd

  Wd1 = blockdiag(W1)
  Wd2 = blockdiag(W2)

  def layer(x, w, root, bias, relu):
    zt = _z_call(x, w)
    zv = zt.reshape(ZG * 16, HA)
    accA, accB = _agg_kernel(zv, iA_r, iB_r, dstr, cidx32, invcnt)
    return _fin_call(accA, accB, x, root, bias, relu)

  h1 = layer(node_emb, Wd1, root1, bias1_2d, relu=True)
  return layer(h1, Wd2, root2, bias2_2d, relu=False)


# speedup vs baseline: 93.5672x; 1.1387x over previous
import functools

import jax
import jax.numpy as jnp
from jax import lax
from jax.experimental import pallas as pl
from jax.experimental.pallas import tpu as pltpu
from jax.experimental.pallas import tpu_sc as plsc

N = 10000
H = 160
HA = 128
HB = 32
R = 50
NB = 5
BD = 32
E = 320000

NC = 2
NS = 16
NW = NC * NS

EPW = E // NW
EPAD = 32
CH = 48
NCH = (EPW + EPAD) // CH
SCI = 11

CCH = 80
CNCH = E // (NS * CCH)

NRP = 512000
CSPAN = NRP // NS
CZCH = 4000

NAP = 10240
NPS = NAP // NS

NT = 2
TN = N // NT
NTF = 10
TF = N // NTF
ZG = R * N // 8


def _cnt_body(cidx_hbm, invcnt_hbm, cidx_v, buf_v, ones_v, cnt_sh):
  c = lax.axis_index("c")
  s = lax.axis_index("s")

  @pl.when(c == 0)
  def _zero():
    def zb(i, _):
      buf_v[pl.ds(i * 16, 16)] = jnp.zeros((16,), jnp.float32)
      return 0
    lax.fori_loop(0, CZCH // 16, zb, 0)
    base = s * CSPAN
    for k in range(CSPAN // CZCH):
      pltpu.sync_copy(buf_v, cnt_sh.at[pl.ds(base + k * CZCH, CZCH)])

  plsc.subcore_barrier()

  @pl.when(c == 0)
  def _count():
    for q in range(CCH // 16):
      ones_v[pl.ds(q * 16, 16)] = jnp.ones((16,), jnp.float32)
    pltpu.sync_copy(cidx_hbm.at[s], cidx_v)

    def body(j, _):
      pltpu.sync_copy(ones_v, cnt_sh.at[cidx_v.at[j]], add=True)
      return 0
    lax.fori_loop(0, CNCH, body, 0)

  plsc.subcore_barrier()

  @pl.when(c == 0)
  def _inv():
    base = s * CSPAN
    for k in range(CSPAN // CZCH):
      pltpu.sync_copy(cnt_sh.at[pl.ds(base + k * CZCH, CZCH)], buf_v)

      def ib(i, _):
        v = buf_v[pl.ds(i * 16, 16)]
        buf_v[pl.ds(i * 16, 16)] = 1.0 / jnp.maximum(v, 1.0)
        return 0
      lax.fori_loop(0, CZCH // 16, ib, 0)
      pltpu.sync_copy(buf_v, invcnt_hbm.at[pl.ds(base + k * CZCH, CZCH)])


_cnt_kernel = pl.kernel(
    _cnt_body,
    out_type=jax.ShapeDtypeStruct((NRP,), jnp.float32),
    mesh=plsc.VectorSubcoreMesh(
        core_axis_name="c", subcore_axis_name="s", num_cores=NC,
        num_subcores=NS),
    scratch_types=[
        pltpu.VMEM((CNCH, CCH), jnp.int32),
        pltpu.VMEM((CZCH,), jnp.float32),
        pltpu.VMEM((CCH,), jnp.float32),
        pltpu.VMEM_SHARED((NRP,), jnp.float32),
    ],
    compiler_params=pltpu.CompilerParams(
        use_tc_tiling_on_sc=False, needs_layout_passes=False),
)


def _agg_body(z_hbm, iA_hbm, iB_hbm, dst_hbm, cidx_hbm, invcnt_hbm,
              accA_hbm, accB_hbm,
              iA_v, iB_v, dst_v, cidx_v, rA0, rA1, rB0, rB1, rC_v, s0, s1,
              accA_sh, accB_sh, semg0, semg1):
  c = lax.axis_index("c")
  s = lax.axis_index("s")
  wid = c * NS + s

  def zba(i, _):
    rA0[i // (HA // 16), pl.ds((i % (HA // 16)) * 16, 16)] = jnp.zeros(
        (16,), jnp.float32)
    return 0
  lax.fori_loop(0, CH * (HA // 16), zba, 0)

  def zbb(i, _):
    rC_v[i // (HB // 16), pl.ds((i % (HB // 16)) * 16, 16)] = jnp.zeros(
        (16,), jnp.float32)
    return 0
  lax.fori_loop(0, CH * (HB // 16), zbb, 0)
  for k in range(NPS // 32):
    pltpu.sync_copy(rA0.at[pl.ds(0, 32)],
                    accA_sh.at[pl.ds(s * NPS + k * 32, 32)])
    pltpu.sync_copy(rC_v.at[pl.ds(0, 32)],
                    accB_sh.at[pl.ds(s * NPS + k * 32, 32)])

  plsc.subcore_barrier()

  rA = (rA0, rA1)
  rB = (rB0, rB1)
  sv_ = (s0, s1)
  sg = (semg0, semg1)

  def body(sc, _):
    pltpu.sync_copy(iA_hbm.at[wid, pl.ds(sc * SCI, SCI)], iA_v)
    pltpu.sync_copy(iB_hbm.at[wid, pl.ds(sc * SCI, SCI)], iB_v)
    pltpu.sync_copy(dst_hbm.at[wid, pl.ds(sc * SCI, SCI)], dst_v)
    pltpu.sync_copy(cidx_hbm.at[wid, pl.ds(sc * SCI, SCI)], cidx_v)

    g = [[None] * 3, [None] * 3]

    def issue(k, p):
      g[p][0] = pltpu.async_copy(z_hbm.at[iA_v.at[k]], rA[p], sg[p])
      g[p][1] = pltpu.async_copy(z_hbm.at[iB_v.at[k]], rB[p], sg[p])
      g[p][2] = pltpu.async_copy(invcnt_hbm.at[cidx_v.at[k]], sv_[p], sg[p])

    issue(0, 0)
    for k in range(SCI):
      p = k % 2
      if k + 1 < SCI:
        issue(k + 1, 1 - p)
      for d in g[p]:
        d.wait()

      def sb(i, _):
        sv = plsc.load_gather(sv_[p], [jnp.full((16,), i, jnp.int32)])
        for q in range(HA // 16):
          rA[p][i, pl.ds(q * 16, 16)] = rA[p][i, pl.ds(q * 16, 16)] * sv
        for q in range(HB // 16):
          rC_v[i, pl.ds(q * 16, 16)] = rB[p][i, pl.ds(q * 16, 16)] * sv
        return 0
      lax.fori_loop(0, CH, sb, 0)

      pltpu.sync_copy(rA[p], accA_sh.at[dst_v.at[k]], add=True)
      pltpu.sync_copy(rC_v, accB_sh.at[dst_v.at[k]], add=True)
    return 0
  lax.fori_loop(0, NCH // SCI, body, 0)

  plsc.subcore_barrier()

  pltpu.sync_copy(accA_sh.at[pl.ds(s * NPS, NPS)],
                  accA_hbm.at[c, pl.ds(s * NPS, NPS)])
  pltpu.sync_copy(accB_sh.at[pl.ds(s * NPS, NPS)],
                  accB_hbm.at[c, pl.ds(s * NPS, NPS)])


_agg_kernel = pl.kernel(
    _agg_body,
    out_type=(jax.ShapeDtypeStruct((NC, NAP, HA), jnp.float32),
              jax.ShapeDtypeStruct((NC, NAP, HB), jnp.float32)),
    mesh=plsc.VectorSubcoreMesh(
        core_axis_name="c", subcore_axis_name="s", num_cores=NC,
        num_subcores=NS),
    scratch_types=[
        pltpu.VMEM((SCI, CH), jnp.int32),
        pltpu.VMEM((SCI, CH), jnp.int32),
        pltpu.VMEM((SCI, CH), jnp.int32),
        pltpu.VMEM((SCI, CH), jnp.int32),
        pltpu.VMEM((CH, HA), jnp.float32),
        pltpu.VMEM((CH, HA), jnp.float32),
        pltpu.VMEM((CH, HA), jnp.float32),
        pltpu.VMEM((CH, HA), jnp.float32),
        pltpu.VMEM((CH, HB), jnp.float32),
        pltpu.VMEM((CH,), jnp.float32),
        pltpu.VMEM((CH,), jnp.float32),
        pltpu.VMEM_SHARED((NAP, HA), jnp.float32),
        pltpu.VMEM_SHARED((NAP, HB), jnp.float32),
        pltpu.SemaphoreType.DMA,
        pltpu.SemaphoreType.DMA,
    ],
    compiler_params=pltpu.CompilerParams(
        use_tc_tiling_on_sc=False, needs_layout_passes=False),
)


def _z_body(x_ref, w_ref, zt_ref):
  y = jnp.dot(x_ref[...], w_ref[0], preferred_element_type=jnp.float32)
  zt_ref[:, 0:8, :] = y[:, 0:HA].reshape(TN // 8, 8, HA)
  zt_ref[:, 8:16, 0:HB] = y[:, HA:H].reshape(TN // 8, 8, HB)


def _z_call(x, wd):
  return pl.pallas_call(
      _z_body,
      grid=(NT, R),
      in_specs=[
          pl.BlockSpec((TN, H), lambda nt, r: (nt, 0)),
          pl.BlockSpec((1, H, H), lambda nt, r: (r, 0, 0)),
      ],
      out_specs=pl.BlockSpec((TN // 8, 16, HA),
                             lambda nt, r: (r * NT + nt, 0, 0)),
      out_shape=jax.ShapeDtypeStruct((ZG, 16, HA), jnp.float32),
  )(x, wd)


def _fin_body(aA_ref, aB_ref, x_ref, root_ref, bias_ref, o_ref, *, relu):
  agg = jnp.concatenate([aA_ref[0] + aA_ref[1], aB_ref[0] + aB_ref[1]],
                        axis=1)
  t = (agg
       + jnp.dot(x_ref[...], root_ref[...],
                 preferred_element_type=jnp.float32)
       + bias_ref[...])
  o_ref[...] = jnp.maximum(t, 0.0) if relu else t


def _fin_call(accA, accB, x, root, bias, relu):
  return pl.pallas_call(
      functools.partial(_fin_body, relu=relu),
      grid=(NTF,),
      in_specs=[
          pl.BlockSpec((NC, TF, HA), lambda nt: (0, nt, 0)),
          pl.BlockSpec((NC, TF, HB), lambda nt: (0, nt, 0)),
          pl.BlockSpec((TF, H), lambda nt: (nt, 0)),
          pl.BlockSpec((H, H), lambda nt: (0, 0)),
          pl.BlockSpec((1, H), lambda nt: (0, 0)),
      ],
      out_specs=pl.BlockSpec((TF, H), lambda nt: (nt, 0)),
      out_shape=jax.ShapeDtypeStruct((N, H), jnp.float32),
  )(accA, accB, x, root, bias)


def kernel(node_emb, edge_index, edge_type, W1, root1, bias1, W2, root2,
           bias2):
  src = edge_index[0].astype(jnp.int32)
  dst = edge_index[1].astype(jnp.int32)
  et = edge_type.astype(jnp.int32)

  g = et * N + src
  iA = (g // 8) * 16 + (g % 8)
  iB = iA + 8
  cidx_flat = dst * R + et

  def wpad(a, padvals):
    pad = jnp.broadcast_to(padvals, (NW, EPAD)).astype(jnp.int32)
    return jnp.concatenate([a.reshape(NW, EPW), pad], axis=1).reshape(
        NW, NCH, CH)

  trash = N + 8 + jnp.arange(EPAD, dtype=jnp.int32)
  iA_r = wpad(iA, jnp.int32(0))
  iB_r = wpad(iB, jnp.int32(8))
  dstr = wpad(dst, trash)
  cidx32 = wpad(cidx_flat, jnp.int32(NRP - 8))
  cidx16 = cidx_flat.reshape(NS, CNCH, CCH)
  bias1_2d = bias1.reshape(1, H)
  bias2_2d = bias2.reshape(1, H)

  invcnt = _cnt_kernel(cidx16)

  def blockdiag(w):
    wd = jnp.zeros((R, H, H), jnp.float32)
    for b in range(NB):
      wd = wd.at[:, b * BD:(b + 1) * BD, b * BD:(b + 1) * BD].set(w[:, b])
    return wd

  Wd1 = blockdiag(W1)
  Wd2 = blockdiag(W2)

  def layer(x, w, root, bias, relu):
    zt = _z_call(x, w)
    zv = zt.reshape(ZG * 16, HA)
    accA, accB = _agg_kernel(zv, iA_r, iB_r, dstr, cidx32, invcnt)
    return _fin_call(accA, accB, x, root, bias, relu)

  h1 = layer(node_emb, Wd1, root1, bias1_2d, relu=True)
  return layer(h1, Wd2, root2, bias2_2d, relu=False)


# speedup vs baseline: 95.6346x; 1.0221x over previous
import functools

import jax
import jax.numpy as jnp
from jax import lax
from jax.experimental import pallas as pl
from jax.experimental.pallas import tpu as pltpu
from jax.experimental.pallas import tpu_sc as plsc

N = 10000
H = 160
HA = 128
HB = 32
R = 50
NB = 5
BD = 32
E = 320000

NC = 2
NS = 16
NW = NC * NS

EPW = E // NW
EPAD = 32
CH = 48
NCH = (EPW + EPAD) // CH
SCI = 11

CCH = 80
CNCH = E // (NS * CCH)

NRP = 512000
CSPAN = NRP // NS
CZCH = 4000

NAP = 10112
NPS = NAP // NS

NT = 2
TN = N // NT
NTF = 10
TF = N // NTF
ZG = R * N // 8


def _cnt_body(cidx_hbm, invcnt_hbm, cidx_v, buf_v, ones_v, cnt_sh):
  c = lax.axis_index("c")
  s = lax.axis_index("s")

  @pl.when(c == 0)
  def _zero():
    def zb(i, _):
      buf_v[pl.ds(i * 16, 16)] = jnp.zeros((16,), jnp.float32)
      return 0
    lax.fori_loop(0, CZCH // 16, zb, 0)
    base = s * CSPAN
    for k in range(CSPAN // CZCH):
      pltpu.sync_copy(buf_v, cnt_sh.at[pl.ds(base + k * CZCH, CZCH)])

  plsc.subcore_barrier()

  @pl.when(c == 0)
  def _count():
    for q in range(CCH // 16):
      ones_v[pl.ds(q * 16, 16)] = jnp.ones((16,), jnp.float32)
    pltpu.sync_copy(cidx_hbm.at[s], cidx_v)

    def body(j, _):
      pltpu.sync_copy(ones_v, cnt_sh.at[cidx_v.at[j]], add=True)
      return 0
    lax.fori_loop(0, CNCH, body, 0)

  plsc.subcore_barrier()

  @pl.when(c == 0)
  def _inv():
    base = s * CSPAN
    for k in range(CSPAN // CZCH):
      pltpu.sync_copy(cnt_sh.at[pl.ds(base + k * CZCH, CZCH)], buf_v)

      def ib(i, _):
        v = buf_v[pl.ds(i * 16, 16)]
        buf_v[pl.ds(i * 16, 16)] = 1.0 / jnp.maximum(v, 1.0)
        return 0
      lax.fori_loop(0, CZCH // 16, ib, 0)
      pltpu.sync_copy(buf_v, invcnt_hbm.at[pl.ds(base + k * CZCH, CZCH)])


_cnt_kernel = pl.kernel(
    _cnt_body,
    out_type=jax.ShapeDtypeStruct((NRP,), jnp.float32),
    mesh=plsc.VectorSubcoreMesh(
        core_axis_name="c", subcore_axis_name="s", num_cores=NC,
        num_subcores=NS),
    scratch_types=[
        pltpu.VMEM((CNCH, CCH), jnp.int32),
        pltpu.VMEM((CZCH,), jnp.float32),
        pltpu.VMEM((CCH,), jnp.float32),
        pltpu.VMEM_SHARED((NRP,), jnp.float32),
    ],
    compiler_params=pltpu.CompilerParams(
        use_tc_tiling_on_sc=False, needs_layout_passes=False),
)


def _agg_body(z_hbm, iA_hbm, iB_hbm, dst_hbm, cidx_hbm, invcnt_hbm,
              accA_hbm, accB_hbm,
              iA_v, iB_v, dst_v, cidx_v, rA0, rA1, rB0, rB1, rC0, rC1, s0, s1,
              accA_sh, accB_sh, semg0, semg1, semc0, semc1):
  c = lax.axis_index("c")
  s = lax.axis_index("s")
  wid = c * NS + s

  def zba(i, _):
    rA0[i // (HA // 16), pl.ds((i % (HA // 16)) * 16, 16)] = jnp.zeros(
        (16,), jnp.float32)
    return 0
  lax.fori_loop(0, CH * (HA // 16), zba, 0)

  def zbb(i, _):
    rC0[i // (HB // 16), pl.ds((i % (HB // 16)) * 16, 16)] = jnp.zeros(
        (16,), jnp.float32)
    return 0
  lax.fori_loop(0, CH * (HB // 16), zbb, 0)
  for k in range(NPS // 32):
    pltpu.sync_copy(rA0.at[pl.ds(0, 32)],
                    accA_sh.at[pl.ds(s * NPS + k * 32, 32)])
    pltpu.sync_copy(rC0.at[pl.ds(0, 32)],
                    accB_sh.at[pl.ds(s * NPS + k * 32, 32)])
  pltpu.sync_copy(rA0.at[pl.ds(0, NPS % 32)],
                  accA_sh.at[pl.ds(s * NPS + (NPS // 32) * 32, NPS % 32)])
  pltpu.sync_copy(rC0.at[pl.ds(0, NPS % 32)],
                  accB_sh.at[pl.ds(s * NPS + (NPS // 32) * 32, NPS % 32)])

  plsc.subcore_barrier()

  rA = (rA0, rA1)
  rB = (rB0, rB1)
  rC = (rC0, rC1)
  sv_ = (s0, s1)
  sg = (semg0, semg1)
  sc_ = (semc0, semc1)

  def body(sc, _):
    pltpu.sync_copy(iA_hbm.at[wid, pl.ds(sc * SCI, SCI)], iA_v)
    pltpu.sync_copy(iB_hbm.at[wid, pl.ds(sc * SCI, SCI)], iB_v)
    pltpu.sync_copy(dst_hbm.at[wid, pl.ds(sc * SCI, SCI)], dst_v)
    pltpu.sync_copy(cidx_hbm.at[wid, pl.ds(sc * SCI, SCI)], cidx_v)

    g = [[None] * 3, [None] * 3]
    scat = [None, None]

    def issue(k, p):
      if scat[p] is not None:
        for d in scat[p]:
          d.wait()
        scat[p] = None
      g[p][0] = pltpu.async_copy(z_hbm.at[iA_v.at[k]], rA[p], sg[p])
      g[p][1] = pltpu.async_copy(z_hbm.at[iB_v.at[k]], rB[p], sg[p])
      g[p][2] = pltpu.async_copy(invcnt_hbm.at[cidx_v.at[k]], sv_[p], sg[p])

    issue(0, 0)
    for k in range(SCI):
      p = k % 2
      if k + 1 < SCI:
        issue(k + 1, 1 - p)
      for d in g[p]:
        d.wait()

      if scat[p] is not None:
        for d in scat[p]:
          d.wait()
        scat[p] = None

      def sb(i, _):
        sv = plsc.load_gather(sv_[p], [jnp.full((16,), i, jnp.int32)])
        for q in range(HA // 16):
          rA[p][i, pl.ds(q * 16, 16)] = rA[p][i, pl.ds(q * 16, 16)] * sv
        for q in range(HB // 16):
          rC[p][i, pl.ds(q * 16, 16)] = rB[p][i, pl.ds(q * 16, 16)] * sv
        return 0
      lax.fori_loop(0, CH, sb, 0)

      scat[p] = [
          pltpu.async_copy(rA[p], accA_sh.at[dst_v.at[k]], sc_[p], add=True),
          pltpu.async_copy(rC[p], accB_sh.at[dst_v.at[k]], sc_[p], add=True),
      ]
    for p in range(2):
      if scat[p] is not None:
        for d in scat[p]:
          d.wait()
    return 0
  lax.fori_loop(0, NCH // SCI, body, 0)

  plsc.subcore_barrier()

  pltpu.sync_copy(accA_sh.at[pl.ds(s * NPS, NPS)],
                  accA_hbm.at[c, pl.ds(s * NPS, NPS)])
  pltpu.sync_copy(accB_sh.at[pl.ds(s * NPS, NPS)],
                  accB_hbm.at[c, pl.ds(s * NPS, NPS)])


_agg_kernel = pl.kernel(
    _agg_body,
    out_type=(jax.ShapeDtypeStruct((NC, NAP, HA), jnp.float32),
              jax.ShapeDtypeStruct((NC, NAP, HB), jnp.float32)),
    mesh=plsc.VectorSubcoreMesh(
        core_axis_name="c", subcore_axis_name="s", num_cores=NC,
        num_subcores=NS),
    scratch_types=[
        pltpu.VMEM((SCI, CH), jnp.int32),
        pltpu.VMEM((SCI, CH), jnp.int32),
        pltpu.VMEM((SCI, CH), jnp.int32),
        pltpu.VMEM((SCI, CH), jnp.int32),
        pltpu.VMEM((CH, HA), jnp.float32),
        pltpu.VMEM((CH, HA), jnp.float32),
        pltpu.VMEM((CH, HA), jnp.float32),
        pltpu.VMEM((CH, HA), jnp.float32),
        pltpu.VMEM((CH, HB), jnp.float32),
        pltpu.VMEM((CH, HB), jnp.float32),
        pltpu.VMEM((CH,), jnp.float32),
        pltpu.VMEM((CH,), jnp.float32),
        pltpu.VMEM_SHARED((NAP, HA), jnp.float32),
        pltpu.VMEM_SHARED((NAP, HB), jnp.float32),
        pltpu.SemaphoreType.DMA,
        pltpu.SemaphoreType.DMA,
        pltpu.SemaphoreType.DMA,
        pltpu.SemaphoreType.DMA,
    ],
    compiler_params=pltpu.CompilerParams(
        use_tc_tiling_on_sc=False, needs_layout_passes=False),
)


def _z_body(x_ref, w_ref, zt_ref):
  y = jnp.dot(x_ref[...], w_ref[0], preferred_element_type=jnp.float32)
  zt_ref[:, 0:8, :] = y[:, 0:HA].reshape(TN // 8, 8, HA)
  zt_ref[:, 8:16, 0:HB] = y[:, HA:H].reshape(TN // 8, 8, HB)


def _z_call(x, wd):
  return pl.pallas_call(
      _z_body,
      grid=(NT, R),
      in_specs=[
          pl.BlockSpec((TN, H), lambda nt, r: (nt, 0)),
          pl.BlockSpec((1, H, H), lambda nt, r: (r, 0, 0)),
      ],
      out_specs=pl.BlockSpec((TN // 8, 16, HA),
                             lambda nt, r: (r * NT + nt, 0, 0)),
      out_shape=jax.ShapeDtypeStruct((ZG, 16, HA), jnp.float32),
  )(x, wd)


def _fin_body(aA_ref, aB_ref, x_ref, root_ref, bias_ref, o_ref, *, relu):
  agg = jnp.concatenate([aA_ref[0] + aA_ref[1], aB_ref[0] + aB_ref[1]],
                        axis=1)
  t = (agg
       + jnp.dot(x_ref[...], root_ref[...],
                 preferred_element_type=jnp.float32)
       + bias_ref[...])
  o_ref[...] = jnp.maximum(t, 0.0) if relu else t


def _fin_call(accA, accB, x, root, bias, relu):
  return pl.pallas_call(
      functools.partial(_fin_body, relu=relu),
      grid=(NTF,),
      in_specs=[
          pl.BlockSpec((NC, TF, HA), lambda nt: (0, nt, 0)),
          pl.BlockSpec((NC, TF, HB), lambda nt: (0, nt, 0)),
          pl.BlockSpec((TF, H), lambda nt: (nt, 0)),
          pl.BlockSpec((H, H), lambda nt: (0, 0)),
          pl.BlockSpec((1, H), lambda nt: (0, 0)),
      ],
      out_specs=pl.BlockSpec((TF, H), lambda nt: (nt, 0)),
      out_shape=jax.ShapeDtypeStruct((N, H), jnp.float32),
  )(accA, accB, x, root, bias)


def kernel(node_emb, edge_index, edge_type, W1, root1, bias1, W2, root2,
           bias2):
  src = edge_index[0].astype(jnp.int32)
  dst = edge_index[1].astype(jnp.int32)
  et = edge_type.astype(jnp.int32)

  g = et * N + src
  iA = (g // 8) * 16 + (g % 8)
  iB = iA + 8
  cidx_flat = dst * R + et

  def wpad(a, padvals):
    pad = jnp.broadcast_to(padvals, (NW, EPAD)).astype(jnp.int32)
    return jnp.concatenate([a.reshape(NW, EPW), pad], axis=1).reshape(
        NW, NCH, CH)

  trash = N + 8 + jnp.arange(EPAD, dtype=jnp.int32)
  iA_r = wpad(iA, jnp.int32(0))
  iB_r = wpad(iB, jnp.int32(8))
  dstr = wpad(dst, trash)
  cidx32 = wpad(cidx_flat, jnp.int32(NRP - 8))
  cidx16 = cidx_flat.reshape(NS, CNCH, CCH)
  bias1_2d = bias1.reshape(1, H)
  bias2_2d = bias2.reshape(1, H)

  invcnt = _cnt_kernel(cidx16)

  def blockdiag(w):
    wd = jnp.zeros((R, H, H), jnp.float32)
    for b in range(NB):
      wd = wd.at[:, b * BD:(b + 1) * BD, b * BD:(b + 1) * BD].set(w[:, b])
    return wd

  Wd1 = blockdiag(W1)
  Wd2 = blockdiag(W2)

  def layer(x, w, root, bias, relu):
    zt = _z_call(x, w)
    zv = zt.reshape(ZG * 16, HA)
    accA, accB = _agg_kernel(zv, iA_r, iB_r, dstr, cidx32, invcnt)
    return _fin_call(accA, accB, x, root, bias, relu)

  h1 = layer(node_emb, Wd1, root1, bias1_2d, relu=True)
  return layer(h1, Wd2, root2, bias2_2d, relu=False)
